# Initial kernel scaffold; baseline (speedup 1.0000x reference)
import jax
import jax.numpy as jnp
from jax.experimental import pallas as pl


def kernel(indices, table):
    raise NotImplementedError("write your pallas kernel here")


# speedup vs baseline: 6.1617x; 6.1617x over previous
import functools

import jax
import jax.numpy as jnp
from jax import lax
from jax.experimental import pallas as pl
from jax.experimental.pallas import tpu as pltpu
from jax.experimental.pallas import tpu_sc as plsc

TOKENS_DIM = 100000
EMBEDDING_DIM = 64
BATCH = 16384
HIST_LEN = 50
_B = BATCH * HIST_LEN

_info = plsc.get_sparse_core_info()
_NC = _info.num_cores
_NS = _info.num_subcores
_NW = _NC * _NS
_BPW = _B // _NW
_CHUNK = 1600
_NCHUNK = _BPW // _CHUNK

_mesh = plsc.VectorSubcoreMesh(core_axis_name="c", subcore_axis_name="s")


@functools.partial(
    pl.kernel,
    mesh=_mesh,
    compiler_params=pltpu.CompilerParams(use_tc_tiling_on_sc=False),
    out_type=jax.ShapeDtypeStruct((_B, EMBEDDING_DIM), jnp.float32),
    scratch_types=[
        pltpu.VMEM((_CHUNK,), jnp.int32),
        pltpu.VMEM((_CHUNK, EMBEDDING_DIM), jnp.float32),
        pltpu.SemaphoreType.DMA,
    ],
)
def _gather_kernel(idx_hbm, table_hbm, out_hbm, idx_v, rows_v, sem):
    wid = lax.axis_index("s") * _NC + lax.axis_index("c")
    base = wid * _BPW

    def body(g, carry):
        off = base + g * _CHUNK
        pltpu.sync_copy(idx_hbm.at[pl.ds(off, _CHUNK)], idx_v)
        pltpu.async_copy(table_hbm.at[idx_v], rows_v, sem).wait()
        pltpu.sync_copy(rows_v, out_hbm.at[pl.ds(off, _CHUNK)])
        return carry

    lax.fori_loop(0, _NCHUNK, body, 0)


def kernel(indices, table):
    flat_idx = indices.reshape(_B).astype(jnp.int32)
    out = _gather_kernel(flat_idx, table)
    return out.reshape(BATCH, HIST_LEN, EMBEDDING_DIM)


# speedup vs baseline: 6.2381x; 1.0124x over previous
import functools

import jax
import jax.numpy as jnp
from jax import lax
from jax.experimental import pallas as pl
from jax.experimental.pallas import tpu as pltpu
from jax.experimental.pallas import tpu_sc as plsc

TOKENS_DIM = 100000
EMBEDDING_DIM = 64
BATCH = 16384
HIST_LEN = 50
_B = BATCH * HIST_LEN

_info = plsc.get_sparse_core_info()
_NC = _info.num_cores
_NS = _info.num_subcores
_NW = _NC * _NS
_BPW = _B // _NW
_CHUNK = 800
_NCHUNK = _BPW // _CHUNK

_mesh = plsc.VectorSubcoreMesh(core_axis_name="c", subcore_axis_name="s")


@functools.partial(
    pl.kernel,
    mesh=_mesh,
    compiler_params=pltpu.CompilerParams(use_tc_tiling_on_sc=False),
    out_type=jax.ShapeDtypeStruct((_B, EMBEDDING_DIM), jnp.float32),
    scratch_types=[
        pltpu.VMEM((_BPW,), jnp.int32),
        pltpu.VMEM((2, _CHUNK, EMBEDDING_DIM), jnp.float32),
        pltpu.SemaphoreType.DMA,
        pltpu.SemaphoreType.DMA,
        pltpu.SemaphoreType.DMA,
        pltpu.SemaphoreType.DMA,
    ],
)
def _gather_kernel(idx_hbm, table_hbm, out_hbm, idx_v, rows_v, g0, g1, s0, s1):
    wid = lax.axis_index("s") * _NC + lax.axis_index("c")
    base = wid * _BPW
    gsem = (g0, g1)
    ssem = (s0, s1)

    pltpu.sync_copy(idx_hbm.at[pl.ds(base, _BPW)], idx_v)

    def start_gather(i, b):
        return pltpu.async_copy(
            table_hbm.at[idx_v.at[pl.ds(i * _CHUNK, _CHUNK)]],
            rows_v.at[b],
            gsem[b],
        )

    def start_store(i, b):
        return pltpu.async_copy(
            rows_v.at[b],
            out_hbm.at[pl.ds(base + i * _CHUNK, _CHUNK)],
            ssem[b],
        )

    start_gather(0, 0)
    start_gather(1, 1)

    def body2(k, carry):
        for b in (0, 1):
            i = 2 * k + b
            pltpu.make_async_copy(
                table_hbm.at[idx_v.at[pl.ds(i * _CHUNK, _CHUNK)]],
                rows_v.at[b],
                gsem[b],
            ).wait()
            start_store(i, b)
            pltpu.make_async_copy(
                rows_v.at[b],
                out_hbm.at[pl.ds(base + i * _CHUNK, _CHUNK)],
                ssem[b],
            ).wait()
            start_gather(i + 2, b)
        return carry

    lax.fori_loop(0, _NCHUNK // 2 - 1, body2, 0)

    last = _NCHUNK - 2
    for b in (0, 1):
        i = last + b
        pltpu.make_async_copy(
            table_hbm.at[idx_v.at[pl.ds(i * _CHUNK, _CHUNK)]],
            rows_v.at[b],
            gsem[b],
        ).wait()
        start_store(i, b).wait()


def kernel(indices, table):
    flat_idx = indices.reshape(_B).astype(jnp.int32)
    out = _gather_kernel(flat_idx, table)
    return out.reshape(BATCH, HIST_LEN, EMBEDDING_DIM)
